# Initial kernel scaffold; baseline (speedup 1.0000x reference)
import jax
import jax.numpy as jnp
from jax.experimental import pallas as pl


def kernel(boxes, scores, hidden_states, labels):
    raise NotImplementedError("write your pallas kernel here")


# speedup vs baseline: 76.5232x; 76.5232x over previous
import jax
import jax.numpy as jnp
from jax.experimental import pallas as pl

_N = 5000
_B = 128
_NB = 40
_NP = _B * _NB
_IOU_T = 0.5
_SCORE_T = 0.2


def _nms_body(x0v_ref, y0v_ref, x1v_ref, y1v_ref, av_ref,
              x0c_ref, y0c_ref, x1c_ref, y1c_ref, ac_ref,
              keep_ref):
    x0v = x0v_ref[...]
    y0v = y0v_ref[...]
    x1v = x1v_ref[...]
    y1v = y1v_ref[...]
    av = av_ref[...]
    x0c = x0c_ref[...]
    y0c = y0c_ref[...]
    x1c = x1c_ref[...]
    y1c = y1c_ref[...]
    ac = ac_ref[...]

    keep_ref[...] = jnp.ones((1, _NP), jnp.float32)

    for i in range(_NB):
        lo = i * _B
        w = _NP - lo
        x0r = jax.lax.slice(x0v, (lo, 0), (lo + _B, 1))
        y0r = jax.lax.slice(y0v, (lo, 0), (lo + _B, 1))
        x1r = jax.lax.slice(x1v, (lo, 0), (lo + _B, 1))
        y1r = jax.lax.slice(y1v, (lo, 0), (lo + _B, 1))
        ar = jax.lax.slice(av, (lo, 0), (lo + _B, 1))

        ltx = jnp.maximum(x0r, jax.lax.slice(x0c, (0, lo), (1, _NP)))
        lty = jnp.maximum(y0r, jax.lax.slice(y0c, (0, lo), (1, _NP)))
        rbx = jnp.minimum(x1r, jax.lax.slice(x1c, (0, lo), (1, _NP)))
        rby = jnp.minimum(y1r, jax.lax.slice(y1c, (0, lo), (1, _NP)))
        ww = jnp.maximum(rbx - ltx, 0.0)
        hh = jnp.maximum(rby - lty, 0.0)
        inter = ww * hh
        union = (ar + jax.lax.slice(ac, (0, lo), (1, _NP))) - inter
        iou = inter / jnp.maximum(union, 1e-9)
        rloc = jax.lax.broadcasted_iota(jnp.int32, (_B, w), 0)
        cloc = jax.lax.broadcasted_iota(jnp.int32, (_B, w), 1)
        s = jnp.where((iou > _IOU_T) & (cloc > rloc), 1.0, 0.0)

        sii = jax.lax.slice(s, (0, 0), (_B, _B))
        alive0 = keep_ref[:, lo:lo + _B]

        def _cond(c):
            return c[1]

        def _body(c):
            k = c[0]
            supp = jax.lax.dot_general(
                k, sii, (((1,), (0,)), ((), ())),
                preferred_element_type=jnp.float32)
            kn = jnp.where(supp > 0.5, 0.0, alive0)
            return kn, jnp.sum(jnp.abs(kn - k)) > 0.0

        kfin, _ = jax.lax.while_loop(
            _cond, _body, (alive0, jnp.bool_(True)))

        supp_all = jax.lax.dot_general(
            kfin, s, (((1,), (0,)), ((), ())),
            preferred_element_type=jnp.float32)
        keep_ref[:, lo:] = jnp.where(supp_all > 0.5, 0.0, keep_ref[:, lo:])


def _mask_body(keep_ref, sc_ref, bx_ref, hd_ref, out_ref):
    sc = sc_ref[...]
    m = jnp.where((keep_ref[...] > 0.5) & (sc >= _SCORE_T), 1.0, 0.0)
    out_ref[...] = jnp.concatenate(
        [sc * m, bx_ref[...] * m, hd_ref[...] * m], axis=1)


def kernel(boxes, scores, hidden_states, labels):
    boxes = boxes.astype(jnp.float32)
    scores = scores.astype(jnp.float32)
    hidden_states = hidden_states.astype(jnp.float32)

    max_coord = jnp.max(boxes)
    offsets = labels.astype(boxes.dtype) * (max_coord + 1.0)
    b = boxes + offsets[:, None]
    order = jnp.argsort(-scores)
    bs = b[order]
    bsp = jnp.concatenate(
        [bs, jnp.zeros((_NP - _N, 4), jnp.float32)], axis=0)
    area = (bsp[:, 2] - bsp[:, 0]) * (bsp[:, 3] - bsp[:, 1])

    x0v = bsp[:, 0:1]
    y0v = bsp[:, 1:2]
    x1v = bsp[:, 2:3]
    y1v = bsp[:, 3:4]
    av = area[:, None]
    x0c = bsp[:, 0][None, :]
    y0c = bsp[:, 1][None, :]
    x1c = bsp[:, 2][None, :]
    y1c = bsp[:, 3][None, :]
    ac = area[None, :]

    keep_s = pl.pallas_call(
        _nms_body,
        out_shape=jax.ShapeDtypeStruct((1, _NP), jnp.float32),
    )(x0v, y0v, x1v, y1v, av, x0c, y0c, x1c, y1c, ac)

    keep = jnp.zeros((_N,), jnp.float32).at[order].set(keep_s[0, :_N])

    rows = 1000
    grid = _N // rows
    out = pl.pallas_call(
        _mask_body,
        grid=(grid,),
        in_specs=[
            pl.BlockSpec((rows, 1), lambda i: (i, 0)),
            pl.BlockSpec((rows, 1), lambda i: (i, 0)),
            pl.BlockSpec((rows, 4), lambda i: (i, 0)),
            pl.BlockSpec((rows, 256), lambda i: (i, 0)),
        ],
        out_specs=pl.BlockSpec((rows, 261), lambda i: (i, 0)),
        out_shape=jax.ShapeDtypeStruct((_N, 261), jnp.float32),
    )(keep[:, None], scores[:, None], boxes, hidden_states)
    return out


# speedup vs baseline: 90.6861x; 1.1851x over previous
import jax
import jax.numpy as jnp
from jax.experimental import pallas as pl

_N = 5000
_B = 128
_NB = 40
_NP = _B * _NB
_SCORE_T = 0.2


def _nms_body(cpk_ref, keep_ref):
    cp = cpk_ref[...]

    rloc = jax.lax.broadcasted_iota(jnp.int32, (_B, _B), 0)
    cloc = jax.lax.broadcasted_iota(jnp.int32, (_B, _B), 1)
    tri = cloc > rloc
    eye = jnp.where(cloc == rloc, 1.0, 0.0)

    keep_ref[...] = jnp.ones((1, _NP), jnp.float32)

    for i in range(_NB):
        lo = i * _B
        hi = lo + _B
        tw = _NP - hi

        xi = jax.lax.slice(cp, (0, lo), (8, hi))
        ti = jax.lax.dot_general(
            eye, xi, (((1,), (1,)), ((), ())),
            precision=jax.lax.Precision.HIGHEST,
            preferred_element_type=jnp.float32)
        x0r = jax.lax.slice(ti, (0, 0), (_B, 1))
        y0r = jax.lax.slice(ti, (0, 1), (_B, 2))
        x1r = jax.lax.slice(ti, (0, 2), (_B, 3))
        y1r = jax.lax.slice(ti, (0, 3), (_B, 4))
        ar = jax.lax.slice(ti, (0, 4), (_B, 5))

        x0d = jax.lax.slice(cp, (0, lo), (1, hi))
        y0d = jax.lax.slice(cp, (1, lo), (2, hi))
        x1d = jax.lax.slice(cp, (2, lo), (3, hi))
        y1d = jax.lax.slice(cp, (3, lo), (4, hi))
        ad = jax.lax.slice(cp, (4, lo), (5, hi))
        ww = jnp.maximum(jnp.minimum(x1r, x1d) - jnp.maximum(x0r, x0d), 0.0)
        hh = jnp.maximum(jnp.minimum(y1r, y1d) - jnp.maximum(y0r, y0d), 0.0)
        inter = ww * hh
        sii = jnp.where((inter * 3.0 > ar + ad) & tri, 1.0, 0.0)

        alive0 = keep_ref[:, lo:hi]

        def _cond(c):
            return c[1]

        def _body(c):
            k = c[0]
            s1 = jax.lax.dot_general(
                k, sii, (((1,), (0,)), ((), ())),
                preferred_element_type=jnp.float32)
            k1 = jnp.where(s1 > 0.5, 0.0, alive0)
            s2 = jax.lax.dot_general(
                k1, sii, (((1,), (0,)), ((), ())),
                preferred_element_type=jnp.float32)
            k2 = jnp.where(s2 > 0.5, 0.0, alive0)
            chg = jnp.sum(jnp.abs(k2 - k1)) > 0.0
            return k2, chg

        kfin, _ = jax.lax.while_loop(
            _cond, _body, (alive0, jnp.bool_(True)))
        keep_ref[:, lo:hi] = kfin

        if tw > 0:
            x0t = jax.lax.slice(cp, (0, hi), (1, _NP))
            y0t = jax.lax.slice(cp, (1, hi), (2, _NP))
            x1t = jax.lax.slice(cp, (2, hi), (3, _NP))
            y1t = jax.lax.slice(cp, (3, hi), (4, _NP))
            at = jax.lax.slice(cp, (4, hi), (5, _NP))
            wt = jnp.maximum(
                jnp.minimum(x1r, x1t) - jnp.maximum(x0r, x0t), 0.0)
            ht = jnp.maximum(
                jnp.minimum(y1r, y1t) - jnp.maximum(y0r, y0t), 0.0)
            it_ = wt * ht
            st = (it_ * 3.0 > ar + at).astype(jnp.bfloat16)
            supp = jax.lax.dot_general(
                kfin.astype(jnp.bfloat16), st, (((1,), (0,)), ((), ())),
                preferred_element_type=jnp.float32)
            keep_ref[:, hi:] = jnp.where(
                supp > 0.5, 0.0, keep_ref[:, hi:])


def _mask_body(p_ref, hd_ref, out_ref):
    p = p_ref[...]
    m = jax.lax.slice(p, (0, 0), (p.shape[0], 1))
    head = jax.lax.slice(p, (0, 1), (p.shape[0], 6))
    out_ref[...] = jnp.concatenate([head, hd_ref[...] * m], axis=1)


def kernel(boxes, scores, hidden_states, labels):
    boxes = boxes.astype(jnp.float32)
    scores = scores.astype(jnp.float32)
    hidden_states = hidden_states.astype(jnp.float32)

    max_coord = jnp.max(boxes)
    offsets = labels.astype(boxes.dtype) * (max_coord + 1.0)
    b = boxes + offsets[:, None]
    order = jnp.argsort(-scores)
    bs = b[order]
    bsp = jnp.concatenate(
        [bs, jnp.zeros((_NP - _N, 4), jnp.float32)], axis=0)
    area = (bsp[:, 2] - bsp[:, 0]) * (bsp[:, 3] - bsp[:, 1])
    cpk = jnp.concatenate(
        [bsp.T, area[None, :], jnp.zeros((3, _NP), jnp.float32)], axis=0)

    keep_s = pl.pallas_call(
        _nms_body,
        out_shape=jax.ShapeDtypeStruct((1, _NP), jnp.float32),
    )(cpk)

    keep = jnp.zeros((_N,), jnp.float32).at[order].set(keep_s[0, :_N])
    m = keep * (scores >= _SCORE_T).astype(jnp.float32)
    p = jnp.concatenate(
        [m[:, None], (scores * m)[:, None], boxes * m[:, None],
         jnp.zeros((_N, 2), jnp.float32)], axis=1)

    rows = 1000
    out = pl.pallas_call(
        _mask_body,
        grid=(_N // rows,),
        in_specs=[
            pl.BlockSpec((rows, 8), lambda i: (i, 0)),
            pl.BlockSpec((rows, 256), lambda i: (i, 0)),
        ],
        out_specs=pl.BlockSpec((rows, 261), lambda i: (i, 0)),
        out_shape=jax.ShapeDtypeStruct((_N, 261), jnp.float32),
    )(p, hidden_states)
    return out


# speedup vs baseline: 103.8620x; 1.1453x over previous
import jax
import jax.numpy as jnp
from jax.experimental import pallas as pl

_N = 5000
_B = 128
_NB = 40
_NP = _B * _NB
_SCORE_T = 0.2


def _nms_body(cpk_ref, keep_ref):
    cp = cpk_ref[...]

    rloc = jax.lax.broadcasted_iota(jnp.int32, (_B, _B), 0)
    cloc = jax.lax.broadcasted_iota(jnp.int32, (_B, _B), 1)
    tri = cloc > rloc
    eye = jnp.where(cloc == rloc, 1.0, 0.0)

    keep_ref[...] = jnp.ones((1, _NP), jnp.float32)

    for i in range(_NB):
        lo = i * _B
        hi = lo + _B
        tw = _NP - hi

        xi = jax.lax.slice(cp, (0, lo), (8, hi))
        ti = jax.lax.dot_general(
            eye, xi, (((1,), (1,)), ((), ())),
            precision=jax.lax.Precision.HIGHEST,
            preferred_element_type=jnp.float32)
        x0r = jax.lax.slice(ti, (0, 0), (_B, 1))
        y0r = jax.lax.slice(ti, (0, 1), (_B, 2))
        x1r = jax.lax.slice(ti, (0, 2), (_B, 3))
        y1r = jax.lax.slice(ti, (0, 3), (_B, 4))
        ar = jax.lax.slice(ti, (0, 4), (_B, 5))

        x0d = jax.lax.slice(cp, (0, lo), (1, hi))
        y0d = jax.lax.slice(cp, (1, lo), (2, hi))
        x1d = jax.lax.slice(cp, (2, lo), (3, hi))
        y1d = jax.lax.slice(cp, (3, lo), (4, hi))
        ad = jax.lax.slice(cp, (4, lo), (5, hi))
        ww = jnp.maximum(jnp.minimum(x1r, x1d) - jnp.maximum(x0r, x0d), 0.0)
        hh = jnp.maximum(jnp.minimum(y1r, y1d) - jnp.maximum(y0r, y0d), 0.0)
        inter = ww * hh
        sii = jnp.where((inter * 3.0 > ar + ad) & tri, 1.0, 0.0)

        alive0 = keep_ref[:, lo:hi]

        def _cond(c):
            return c[1]

        def _body(c):
            k = c[0]
            s1 = jax.lax.dot_general(
                k, sii, (((1,), (0,)), ((), ())),
                preferred_element_type=jnp.float32)
            k1 = jnp.where(s1 > 0.5, 0.0, alive0)
            s2 = jax.lax.dot_general(
                k1, sii, (((1,), (0,)), ((), ())),
                preferred_element_type=jnp.float32)
            k2 = jnp.where(s2 > 0.5, 0.0, alive0)
            chg = jnp.sum(jnp.abs(k2 - k1)) > 0.0
            return k2, chg

        kfin, _ = jax.lax.while_loop(
            _cond, _body, (alive0, jnp.bool_(True)))
        keep_ref[:, lo:hi] = kfin

        if tw > 0:
            x0t = jax.lax.slice(cp, (0, hi), (1, _NP))
            y0t = jax.lax.slice(cp, (1, hi), (2, _NP))
            x1t = jax.lax.slice(cp, (2, hi), (3, _NP))
            y1t = jax.lax.slice(cp, (3, hi), (4, _NP))
            at = jax.lax.slice(cp, (4, hi), (5, _NP))
            wt = jnp.maximum(
                jnp.minimum(x1r, x1t) - jnp.maximum(x0r, x0t), 0.0)
            ht = jnp.maximum(
                jnp.minimum(y1r, y1t) - jnp.maximum(y0r, y0t), 0.0)
            it_ = wt * ht
            st = (it_ * 3.0 > ar + at).astype(jnp.bfloat16)
            supp = jax.lax.dot_general(
                kfin.astype(jnp.bfloat16), st, (((1,), (0,)), ((), ())),
                preferred_element_type=jnp.float32)
            keep_ref[:, hi:] = jnp.where(
                supp > 0.5, 0.0, keep_ref[:, hi:])


def _mask_body(p_ref, hd_ref, out_ref):
    p = p_ref[...]
    m = jax.lax.slice(p, (0, 0), (p.shape[0], 1))
    head = jax.lax.slice(p, (0, 1), (p.shape[0], 6))
    out_ref[...] = jnp.concatenate([head, hd_ref[...] * m], axis=1)


def kernel(boxes, scores, hidden_states, labels):
    boxes = boxes.astype(jnp.float32)
    scores = scores.astype(jnp.float32)
    hidden_states = hidden_states.astype(jnp.float32)

    max_coord = jnp.max(boxes)
    offsets = labels.astype(boxes.dtype) * (max_coord + 1.0)
    b = boxes + offsets[:, None]
    order = jnp.argsort(-scores)
    bs = b[order]
    bsp = jnp.concatenate(
        [bs, jnp.zeros((_NP - _N, 4), jnp.float32)], axis=0)
    area = (bsp[:, 2] - bsp[:, 0]) * (bsp[:, 3] - bsp[:, 1])
    cpk = jnp.concatenate(
        [bsp.T, area[None, :], jnp.zeros((3, _NP), jnp.float32)], axis=0)

    keep_s = pl.pallas_call(
        _nms_body,
        out_shape=jax.ShapeDtypeStruct((1, _NP), jnp.float32),
    )(cpk)

    keep = jax.lax.sort_key_val(
        order.astype(jnp.int32), keep_s[0, :_N])[1]
    m = keep * (scores >= _SCORE_T).astype(jnp.float32)
    p = jnp.concatenate(
        [m[:, None], (scores * m)[:, None], boxes * m[:, None],
         jnp.zeros((_N, 2), jnp.float32)], axis=1)

    rows = 1000
    out = pl.pallas_call(
        _mask_body,
        grid=(_N // rows,),
        in_specs=[
            pl.BlockSpec((rows, 8), lambda i: (i, 0)),
            pl.BlockSpec((rows, 256), lambda i: (i, 0)),
        ],
        out_specs=pl.BlockSpec((rows, 261), lambda i: (i, 0)),
        out_shape=jax.ShapeDtypeStruct((_N, 261), jnp.float32),
    )(p, hidden_states)
    return out


# speedup vs baseline: 132.0212x; 1.2711x over previous
import jax
import jax.numpy as jnp
from jax.experimental import pallas as pl

_N = 5000
_B = 128
_NB = 40
_NP = _B * _NB
_SCORE_T = 0.2


def _nms_body(cpk_ref, keep_ref):
    cp = cpk_ref[...]

    rloc = jax.lax.broadcasted_iota(jnp.int32, (_B, _B), 0)
    cloc = jax.lax.broadcasted_iota(jnp.int32, (_B, _B), 1)
    tri = cloc > rloc
    eye = jnp.where(cloc == rloc, 1.0, 0.0)

    keep_ref[...] = jnp.ones((1, _NP), jnp.float32)

    for i in range(_NB):
        lo = i * _B
        hi = lo + _B
        tw = _NP - hi

        xi = jax.lax.slice(cp, (0, lo), (8, hi))
        ti = jax.lax.dot_general(
            eye, xi, (((1,), (1,)), ((), ())),
            precision=jax.lax.Precision.HIGHEST,
            preferred_element_type=jnp.float32)
        x0r = jax.lax.slice(ti, (0, 0), (_B, 1))
        y0r = jax.lax.slice(ti, (0, 1), (_B, 2))
        x1r = jax.lax.slice(ti, (0, 2), (_B, 3))
        y1r = jax.lax.slice(ti, (0, 3), (_B, 4))
        ar = jax.lax.slice(ti, (0, 4), (_B, 5))

        x0d = jax.lax.slice(cp, (0, lo), (1, hi))
        y0d = jax.lax.slice(cp, (1, lo), (2, hi))
        x1d = jax.lax.slice(cp, (2, lo), (3, hi))
        y1d = jax.lax.slice(cp, (3, lo), (4, hi))
        ad = jax.lax.slice(cp, (4, lo), (5, hi))
        ww = jnp.maximum(jnp.minimum(x1r, x1d) - jnp.maximum(x0r, x0d), 0.0)
        hh = jnp.maximum(jnp.minimum(y1r, y1d) - jnp.maximum(y0r, y0d), 0.0)
        inter = ww * hh
        sii = jnp.where((inter * 3.0 > ar + ad) & tri, 1.0, 0.0)

        alive0 = keep_ref[:, lo:hi]

        def _cond(c):
            return c[1]

        def _body(c):
            k = c[0]
            s1 = jax.lax.dot_general(
                k, sii, (((1,), (0,)), ((), ())),
                preferred_element_type=jnp.float32)
            k1 = jnp.where(s1 > 0.5, 0.0, alive0)
            s2 = jax.lax.dot_general(
                k1, sii, (((1,), (0,)), ((), ())),
                preferred_element_type=jnp.float32)
            k2 = jnp.where(s2 > 0.5, 0.0, alive0)
            chg = jnp.sum(jnp.abs(k2 - k1)) > 0.0
            return k2, chg

        kfin, _ = jax.lax.while_loop(
            _cond, _body, (alive0, jnp.bool_(True)))
        keep_ref[:, lo:hi] = kfin

        if tw > 0:
            x0t = jax.lax.slice(cp, (0, hi), (1, _NP))
            y0t = jax.lax.slice(cp, (1, hi), (2, _NP))
            x1t = jax.lax.slice(cp, (2, hi), (3, _NP))
            y1t = jax.lax.slice(cp, (3, hi), (4, _NP))
            at = jax.lax.slice(cp, (4, hi), (5, _NP))
            wt = jnp.maximum(
                jnp.minimum(x1r, x1t) - jnp.maximum(x0r, x0t), 0.0)
            ht = jnp.maximum(
                jnp.minimum(y1r, y1t) - jnp.maximum(y0r, y0t), 0.0)
            it_ = wt * ht
            st = (it_ * 3.0 > ar + at).astype(jnp.bfloat16)
            supp = jax.lax.dot_general(
                kfin.astype(jnp.bfloat16), st, (((1,), (0,)), ((), ())),
                preferred_element_type=jnp.float32)
            keep_ref[:, hi:] = jnp.where(
                supp > 0.5, 0.0, keep_ref[:, hi:])


def _mask_body(p_ref, hd_ref, out_ref):
    p = p_ref[...]
    m = jax.lax.slice(p, (0, 0), (p.shape[0], 1))
    head = jax.lax.slice(p, (0, 1), (p.shape[0], 6))
    out_ref[...] = jnp.concatenate([head, hd_ref[...] * m], axis=1)


def kernel(boxes, scores, hidden_states, labels):
    boxes = boxes.astype(jnp.float32)
    scores = scores.astype(jnp.float32)
    hidden_states = hidden_states.astype(jnp.float32)

    max_coord = jnp.max(boxes)
    offsets = labels.astype(boxes.dtype) * (max_coord + 1.0)
    b = boxes + offsets[:, None]
    _, order, x0s, y0s, x1s, y1s = jax.lax.sort(
        (-scores, jnp.arange(_N, dtype=jnp.int32),
         b[:, 0], b[:, 1], b[:, 2], b[:, 3]), num_keys=1)
    pad = jnp.zeros((_NP - _N,), jnp.float32)
    x0p = jnp.concatenate([x0s, pad])
    y0p = jnp.concatenate([y0s, pad])
    x1p = jnp.concatenate([x1s, pad])
    y1p = jnp.concatenate([y1s, pad])
    area = (x1p - x0p) * (y1p - y0p)
    cpk = jnp.stack(
        [x0p, y0p, x1p, y1p, area,
         jnp.zeros((_NP,), jnp.float32),
         jnp.zeros((_NP,), jnp.float32),
         jnp.zeros((_NP,), jnp.float32)], axis=0)

    keep_s = pl.pallas_call(
        _nms_body,
        out_shape=jax.ShapeDtypeStruct((1, _NP), jnp.float32),
    )(cpk)

    keep = jax.lax.sort_key_val(
        order.astype(jnp.int32), keep_s[0, :_N])[1]
    m = keep * (scores >= _SCORE_T).astype(jnp.float32)
    p = jnp.concatenate(
        [m[:, None], (scores * m)[:, None], boxes * m[:, None],
         jnp.zeros((_N, 2), jnp.float32)], axis=1)

    rows = 1000
    out = pl.pallas_call(
        _mask_body,
        grid=(_N // rows,),
        in_specs=[
            pl.BlockSpec((rows, 8), lambda i: (i, 0)),
            pl.BlockSpec((rows, 256), lambda i: (i, 0)),
        ],
        out_specs=pl.BlockSpec((rows, 261), lambda i: (i, 0)),
        out_shape=jax.ShapeDtypeStruct((_N, 261), jnp.float32),
    )(p, hidden_states)
    return out


# speedup vs baseline: 142.7548x; 1.0813x over previous
import jax
import jax.numpy as jnp
from jax.experimental import pallas as pl

_N = 5000
_B = 256
_NB = 20
_NP = _B * _NB
_SCORE_T = 0.2


def _nms_body(cpk_ref, keep_ref):
    cp = cpk_ref[...]

    rloc = jax.lax.broadcasted_iota(jnp.int32, (_B, _B), 0)
    cloc = jax.lax.broadcasted_iota(jnp.int32, (_B, _B), 1)
    tri = cloc > rloc
    eye = jnp.where(cloc == rloc, 1.0, 0.0)

    keep_ref[...] = jnp.ones((1, _NP), jnp.float32)

    for i in range(_NB):
        lo = i * _B
        hi = lo + _B
        tw = _NP - hi

        xi = jax.lax.slice(cp, (0, lo), (8, hi))
        ti = jax.lax.dot_general(
            eye, xi, (((1,), (1,)), ((), ())),
            precision=jax.lax.Precision.HIGHEST,
            preferred_element_type=jnp.float32)
        x0r = jax.lax.slice(ti, (0, 0), (_B, 1))
        y0r = jax.lax.slice(ti, (0, 1), (_B, 2))
        x1r = jax.lax.slice(ti, (0, 2), (_B, 3))
        y1r = jax.lax.slice(ti, (0, 3), (_B, 4))
        ar = jax.lax.slice(ti, (0, 4), (_B, 5))

        x0d = jax.lax.slice(cp, (0, lo), (1, hi))
        y0d = jax.lax.slice(cp, (1, lo), (2, hi))
        x1d = jax.lax.slice(cp, (2, lo), (3, hi))
        y1d = jax.lax.slice(cp, (3, lo), (4, hi))
        ad = jax.lax.slice(cp, (4, lo), (5, hi))
        ww = jnp.maximum(jnp.minimum(x1r, x1d) - jnp.maximum(x0r, x0d), 0.0)
        hh = jnp.maximum(jnp.minimum(y1r, y1d) - jnp.maximum(y0r, y0d), 0.0)
        inter = ww * hh
        sii = jnp.where((inter * 3.0 > ar + ad) & tri, 1.0, 0.0)

        alive0 = keep_ref[:, lo:hi]

        def _cond(c):
            return c[1]

        def _body(c):
            k = c[0]
            s1 = jax.lax.dot_general(
                k, sii, (((1,), (0,)), ((), ())),
                preferred_element_type=jnp.float32)
            k1 = jnp.where(s1 > 0.5, 0.0, alive0)
            s2 = jax.lax.dot_general(
                k1, sii, (((1,), (0,)), ((), ())),
                preferred_element_type=jnp.float32)
            k2 = jnp.where(s2 > 0.5, 0.0, alive0)
            chg = jnp.sum(jnp.abs(k2 - k1)) > 0.0
            return k2, chg

        kfin, _ = jax.lax.while_loop(
            _cond, _body, (alive0, jnp.bool_(True)))
        keep_ref[:, lo:hi] = kfin

        if tw > 0:
            x0t = jax.lax.slice(cp, (0, hi), (1, _NP))
            y0t = jax.lax.slice(cp, (1, hi), (2, _NP))
            x1t = jax.lax.slice(cp, (2, hi), (3, _NP))
            y1t = jax.lax.slice(cp, (3, hi), (4, _NP))
            at = jax.lax.slice(cp, (4, hi), (5, _NP))
            wt = jnp.maximum(
                jnp.minimum(x1r, x1t) - jnp.maximum(x0r, x0t), 0.0)
            ht = jnp.maximum(
                jnp.minimum(y1r, y1t) - jnp.maximum(y0r, y0t), 0.0)
            it_ = wt * ht
            st = (it_ * 3.0 > ar + at).astype(jnp.bfloat16)
            supp = jax.lax.dot_general(
                kfin.astype(jnp.bfloat16), st, (((1,), (0,)), ((), ())),
                preferred_element_type=jnp.float32)
            keep_ref[:, hi:] = jnp.where(
                supp > 0.5, 0.0, keep_ref[:, hi:])


def _mask_body(p_ref, hd_ref, out_ref):
    p = p_ref[...]
    m = jax.lax.slice(p, (0, 0), (p.shape[0], 1))
    head = jax.lax.slice(p, (0, 1), (p.shape[0], 6))
    out_ref[...] = jnp.concatenate([head, hd_ref[...] * m], axis=1)


def kernel(boxes, scores, hidden_states, labels):
    boxes = boxes.astype(jnp.float32)
    scores = scores.astype(jnp.float32)
    hidden_states = hidden_states.astype(jnp.float32)

    max_coord = jnp.max(boxes)
    offsets = labels.astype(boxes.dtype) * (max_coord + 1.0)
    b = boxes + offsets[:, None]
    _, order, x0s, y0s, x1s, y1s = jax.lax.sort(
        (-scores, jnp.arange(_N, dtype=jnp.int32),
         b[:, 0], b[:, 1], b[:, 2], b[:, 3]), num_keys=1)
    pad = jnp.zeros((_NP - _N,), jnp.float32)
    x0p = jnp.concatenate([x0s, pad])
    y0p = jnp.concatenate([y0s, pad])
    x1p = jnp.concatenate([x1s, pad])
    y1p = jnp.concatenate([y1s, pad])
    area = (x1p - x0p) * (y1p - y0p)
    cpk = jnp.stack(
        [x0p, y0p, x1p, y1p, area,
         jnp.zeros((_NP,), jnp.float32),
         jnp.zeros((_NP,), jnp.float32),
         jnp.zeros((_NP,), jnp.float32)], axis=0)

    keep_s = pl.pallas_call(
        _nms_body,
        out_shape=jax.ShapeDtypeStruct((1, _NP), jnp.float32),
    )(cpk)

    keep = jax.lax.sort_key_val(
        order.astype(jnp.int32), keep_s[0, :_N])[1]
    m = keep * (scores >= _SCORE_T).astype(jnp.float32)
    p = jnp.concatenate(
        [m[:, None], (scores * m)[:, None], boxes * m[:, None],
         jnp.zeros((_N, 2), jnp.float32)], axis=1)

    rows = 1000
    out = pl.pallas_call(
        _mask_body,
        grid=(_N // rows,),
        in_specs=[
            pl.BlockSpec((rows, 8), lambda i: (i, 0)),
            pl.BlockSpec((rows, 256), lambda i: (i, 0)),
        ],
        out_specs=pl.BlockSpec((rows, 261), lambda i: (i, 0)),
        out_shape=jax.ShapeDtypeStruct((_N, 261), jnp.float32),
    )(p, hidden_states)
    return out


# speedup vs baseline: 144.1722x; 1.0099x over previous
import jax
import jax.numpy as jnp
from jax.experimental import pallas as pl

_N = 5000
_B = 512
_NB = 10
_NP = _B * _NB
_SCORE_T = 0.2


def _nms_body(cpk_ref, keep_ref):
    cp = cpk_ref[...]

    rloc = jax.lax.broadcasted_iota(jnp.int32, (_B, _B), 0)
    cloc = jax.lax.broadcasted_iota(jnp.int32, (_B, _B), 1)
    tri = cloc > rloc
    eye = jnp.where(cloc == rloc, 1.0, 0.0)

    keep_ref[...] = jnp.ones((1, _NP), jnp.float32)

    for i in range(_NB):
        lo = i * _B
        hi = lo + _B
        tw = _NP - hi

        xi = jax.lax.slice(cp, (0, lo), (8, hi))
        ti = jax.lax.dot_general(
            eye, xi, (((1,), (1,)), ((), ())),
            precision=jax.lax.Precision.HIGHEST,
            preferred_element_type=jnp.float32)
        x0r = jax.lax.slice(ti, (0, 0), (_B, 1))
        y0r = jax.lax.slice(ti, (0, 1), (_B, 2))
        x1r = jax.lax.slice(ti, (0, 2), (_B, 3))
        y1r = jax.lax.slice(ti, (0, 3), (_B, 4))
        ar = jax.lax.slice(ti, (0, 4), (_B, 5))

        x0d = jax.lax.slice(cp, (0, lo), (1, hi))
        y0d = jax.lax.slice(cp, (1, lo), (2, hi))
        x1d = jax.lax.slice(cp, (2, lo), (3, hi))
        y1d = jax.lax.slice(cp, (3, lo), (4, hi))
        ad = jax.lax.slice(cp, (4, lo), (5, hi))
        ww = jnp.maximum(jnp.minimum(x1r, x1d) - jnp.maximum(x0r, x0d), 0.0)
        hh = jnp.maximum(jnp.minimum(y1r, y1d) - jnp.maximum(y0r, y0d), 0.0)
        inter = ww * hh
        sii = jnp.where((inter * 3.0 > ar + ad) & tri, 1.0, 0.0)

        alive0 = keep_ref[:, lo:hi]

        def _cond(c):
            return c[1]

        def _body(c):
            k = c[0]
            s1 = jax.lax.dot_general(
                k, sii, (((1,), (0,)), ((), ())),
                preferred_element_type=jnp.float32)
            k1 = jnp.where(s1 > 0.5, 0.0, alive0)
            s2 = jax.lax.dot_general(
                k1, sii, (((1,), (0,)), ((), ())),
                preferred_element_type=jnp.float32)
            k2 = jnp.where(s2 > 0.5, 0.0, alive0)
            chg = jnp.sum(jnp.abs(k2 - k1)) > 0.0
            return k2, chg

        kfin, _ = jax.lax.while_loop(
            _cond, _body, (alive0, jnp.bool_(True)))
        keep_ref[:, lo:hi] = kfin

        if tw > 0:
            x0t = jax.lax.slice(cp, (0, hi), (1, _NP))
            y0t = jax.lax.slice(cp, (1, hi), (2, _NP))
            x1t = jax.lax.slice(cp, (2, hi), (3, _NP))
            y1t = jax.lax.slice(cp, (3, hi), (4, _NP))
            at = jax.lax.slice(cp, (4, hi), (5, _NP))
            wt = jnp.maximum(
                jnp.minimum(x1r, x1t) - jnp.maximum(x0r, x0t), 0.0)
            ht = jnp.maximum(
                jnp.minimum(y1r, y1t) - jnp.maximum(y0r, y0t), 0.0)
            it_ = wt * ht
            st = (it_ * 3.0 > ar + at).astype(jnp.bfloat16)
            supp = jax.lax.dot_general(
                kfin.astype(jnp.bfloat16), st, (((1,), (0,)), ((), ())),
                preferred_element_type=jnp.float32)
            keep_ref[:, hi:] = jnp.where(
                supp > 0.5, 0.0, keep_ref[:, hi:])


def _mask_body(p_ref, hd_ref, out_ref):
    p = p_ref[...]
    m = jax.lax.slice(p, (0, 0), (p.shape[0], 1))
    head = jax.lax.slice(p, (0, 1), (p.shape[0], 6))
    out_ref[...] = jnp.concatenate([head, hd_ref[...] * m], axis=1)


def kernel(boxes, scores, hidden_states, labels):
    boxes = boxes.astype(jnp.float32)
    scores = scores.astype(jnp.float32)
    hidden_states = hidden_states.astype(jnp.float32)

    max_coord = jnp.max(boxes)
    offsets = labels.astype(boxes.dtype) * (max_coord + 1.0)
    b = boxes + offsets[:, None]
    _, order, x0s, y0s, x1s, y1s = jax.lax.sort(
        (-scores, jnp.arange(_N, dtype=jnp.int32),
         b[:, 0], b[:, 1], b[:, 2], b[:, 3]), num_keys=1)
    pad = jnp.zeros((_NP - _N,), jnp.float32)
    x0p = jnp.concatenate([x0s, pad])
    y0p = jnp.concatenate([y0s, pad])
    x1p = jnp.concatenate([x1s, pad])
    y1p = jnp.concatenate([y1s, pad])
    area = (x1p - x0p) * (y1p - y0p)
    cpk = jnp.stack(
        [x0p, y0p, x1p, y1p, area,
         jnp.zeros((_NP,), jnp.float32),
         jnp.zeros((_NP,), jnp.float32),
         jnp.zeros((_NP,), jnp.float32)], axis=0)

    keep_s = pl.pallas_call(
        _nms_body,
        out_shape=jax.ShapeDtypeStruct((1, _NP), jnp.float32),
    )(cpk)

    keep = jax.lax.sort_key_val(
        order.astype(jnp.int32), keep_s[0, :_N])[1]
    m = keep * (scores >= _SCORE_T).astype(jnp.float32)
    p = jnp.concatenate(
        [m[:, None], (scores * m)[:, None], boxes * m[:, None],
         jnp.zeros((_N, 2), jnp.float32)], axis=1)

    rows = 1000
    out = pl.pallas_call(
        _mask_body,
        grid=(_N // rows,),
        in_specs=[
            pl.BlockSpec((rows, 8), lambda i: (i, 0)),
            pl.BlockSpec((rows, 256), lambda i: (i, 0)),
        ],
        out_specs=pl.BlockSpec((rows, 261), lambda i: (i, 0)),
        out_shape=jax.ShapeDtypeStruct((_N, 261), jnp.float32),
    )(p, hidden_states)
    return out
